# baseline (speedup 1.0000x reference)
import functools

import jax
import jax.numpy as jnp
from jax import lax
from jax.experimental import pallas as pl
from jax.experimental.pallas import tpu as pltpu
from jax.experimental.pallas import tpu_sc as plsc

C = 64
H = 64
W = 64
HW = H * W
P2 = 9
D2 = 9
F32 = jnp.float32

NW = 32
PXW = HW // NW
L = 16


def _wmask(v):
    w = lax.broadcasted_iota(jnp.int32, (1, HW), 1) % W
    return ((w + v >= 0) & (w + v < W)).astype(F32)


def _fshift(x, u, v):
    s = u * W + v
    n = x.shape[0]
    if s > 0:
        y = jnp.concatenate([x[:, s:], jnp.zeros((n, s), x.dtype)], axis=1)
    elif s < 0:
        y = jnp.concatenate([jnp.zeros((n, -s), x.dtype), x[:, :s]], axis=1)
    else:
        y = x
    if v != 0:
        y = y * _wmask(v)
    return y


def _boxsum(x):
    r = x + _fshift(x, 0, 1) + _fshift(x, 0, -1)
    return r + _fshift(r, 1, 0) + _fshift(r, -1, 0)


def _valid(dy, dx):
    hh = lax.broadcasted_iota(jnp.int32, (1, HW), 1) // W
    ww = lax.broadcasted_iota(jnp.int32, (1, HW), 1) % W
    return (hh + dy >= 0) & (hh + dy < H) & (ww + dx >= 0) & (ww + dx < W)


def _sims_body(ft_ref, fm_ref, aw_ref, sims_ref, awb_ref):
    for r in range(4):
        awb_ref[r, :] = jnp.full((L,), aw_ref[r], F32)
    ft = ft_ref[...]
    fm = fm_ref[...]
    na = jnp.maximum(jnp.sqrt(_boxsum(jnp.sum(ft * ft, axis=0, keepdims=True))), 1e-12)
    nb = jnp.maximum(jnp.sqrt(_boxsum(jnp.sum(fm * fm, axis=0, keepdims=True))), 1e-12)
    for di in range(3):
        for dj in range(3):
            dy, dx = di - 1, dj - 1
            corr = jnp.sum(ft * _fshift(fm, dy, dx), axis=0, keepdims=True)
            raw = _boxsum(corr)
            nbs = jnp.maximum(_fshift(nb, dy, dx), 1e-12)
            sim = jnp.where(_valid(dy, dx), raw / (na * nbs), 0.0)
            sims_ref[pl.ds(di * 3 + dj, 1), :] = sim


def _topk_body(sims_hbm, awb_hbm, a_hbm, sims_v, a_v, awv):
    wid = lax.axis_index("s") * 2 + lax.axis_index("c")
    base = wid * PXW
    pltpu.sync_copy(sims_hbm.at[:, pl.ds(base, PXW)], sims_v)
    pltpu.sync_copy(awb_hbm, awv)
    for g in range(PXW // L):
        cur = [sims_v[dd, pl.ds(g * L, L)] for dd in range(D2)]
        amaps = [jnp.zeros((L,), F32) for _ in range(D2)]
        for r in range(4):
            aw_r = awv[r, :]
            best = cur[0]
            bidx = jnp.zeros((L,), jnp.int32)
            for dd in range(1, D2):
                cond = cur[dd] > best
                best = jnp.where(cond, cur[dd], best)
                bidx = jnp.where(cond, dd, bidx)
            for dd in range(D2):
                amaps[dd] = amaps[dd] + jnp.where(bidx == dd, aw_r, 0.0)
            cur = [jnp.where(bidx == dd, -5.0, cur[dd]) for dd in range(D2)]
        for dd in range(D2):
            a_v[dd, pl.ds(g * L, L)] = amaps[dd]
    pltpu.sync_copy(a_v, a_hbm.at[:, pl.ds(base, PXW)])


def _agg_body(ft_ref, fm_ref, w3_ref, pb3_ref, a_ref, ab_ref, out_ref, fms_ref):
    fm = fm_ref[...]
    for u in range(-2, 3):
        for v in range(-2, 3):
            fms_ref[(u + 2) * 5 + (v + 2)] = _fshift(fm, u, v)
    ab = ab_ref[0]
    a_b = []
    for di in range(3):
        for dj in range(3):
            dd = di * 3 + dj
            a_dd = a_ref[pl.ds(dd, 1), :] * _valid(di - 1, dj - 1).astype(F32)
            a_b.append(jnp.broadcast_to(a_dd, (C, HW)))
    acc = jnp.zeros((C, HW), F32)
    for pp in range(P2):
        pi, pj = pp // 3, pp % 3
        w_pp = w3_ref[:, pp, :]
        pb_pp = pb3_ref[:, pp, :]
        wp_pp = (jnp.dot(w_pp[:, 0:C], ft_ref[...], preferred_element_type=F32)
                 + jnp.dot(w_pp[:, C:2 * C], fm, preferred_element_type=F32)
                 + pb_pp)
        b_pp = jnp.zeros((C, HW), F32)
        for di in range(3):
            for dj in range(3):
                dd = di * 3 + dj
                sel = (di - 1 + pi - 1 + 2) * 5 + (dj - 1 + pj - 1 + 2)
                b_pp = b_pp + a_b[dd] * fms_ref[sel]
        acc = acc + wp_pp * (b_pp + ab)
    out_ref[...] = acc


@functools.partial(jax.jit, static_argnames=())
def kernel(feat_t, feat_tm1, agg_w, agg_b, proj_w, proj_b):
    ft = feat_t.reshape(C, HW)
    fm = feat_tm1.reshape(C, HW)
    w3 = proj_w.reshape(C, P2, 2 * C)
    pb3 = proj_b.reshape(C, P2, 1)
    ab = agg_b.reshape(1)

    sims, awb = pl.pallas_call(
        _sims_body,
        out_shape=(jax.ShapeDtypeStruct((D2, HW), F32),
                   jax.ShapeDtypeStruct((4, L), F32)),
        in_specs=[pl.BlockSpec(memory_space=pltpu.VMEM),
                  pl.BlockSpec(memory_space=pltpu.VMEM),
                  pl.BlockSpec(memory_space=pltpu.SMEM)],
    )(ft, fm, agg_w.reshape(4))

    sc_topk = pl.kernel(
        _topk_body,
        out_type=jax.ShapeDtypeStruct((D2, HW), F32),
        mesh=plsc.VectorSubcoreMesh(core_axis_name="c", subcore_axis_name="s"),
        scratch_types=[
            pltpu.VMEM((D2, PXW), F32),
            pltpu.VMEM((D2, PXW), F32),
            pltpu.VMEM((4, L), F32),
        ],
    )
    amap = sc_topk(sims, awb)

    out = pl.pallas_call(
        _agg_body,
        out_shape=jax.ShapeDtypeStruct((C, HW), F32),
        in_specs=[
            pl.BlockSpec(memory_space=pltpu.VMEM),
            pl.BlockSpec(memory_space=pltpu.VMEM),
            pl.BlockSpec(memory_space=pltpu.VMEM),
            pl.BlockSpec(memory_space=pltpu.VMEM),
            pl.BlockSpec(memory_space=pltpu.VMEM),
            pl.BlockSpec(memory_space=pltpu.SMEM),
        ],
        scratch_shapes=[pltpu.VMEM((25, C, HW), F32)],
    )(ft, fm, w3, pb3, amap, ab)
    return out.reshape(1, C, H, W)


# speedup vs baseline: 1.0199x; 1.0199x over previous
import functools

import jax
import jax.numpy as jnp
from jax import lax
from jax.experimental import pallas as pl
from jax.experimental.pallas import tpu as pltpu
from jax.experimental.pallas import tpu_sc as plsc

C = 64
H = 64
W = 64
HW = H * W
P2 = 9
D2 = 9
F32 = jnp.float32

NW = 32
PXW = HW // NW
L = 16


def _wmask(v):
    w = lax.broadcasted_iota(jnp.int32, (1, HW), 1) % W
    return ((w + v >= 0) & (w + v < W)).astype(F32)


def _fshift(x, u, v):
    s = u * W + v
    n = x.shape[0]
    if s > 0:
        y = jnp.concatenate([x[:, s:], jnp.zeros((n, s), x.dtype)], axis=1)
    elif s < 0:
        y = jnp.concatenate([jnp.zeros((n, -s), x.dtype), x[:, :s]], axis=1)
    else:
        y = x
    if v != 0:
        y = y * _wmask(v)
    return y


def _boxsum(x):
    r = x + _fshift(x, 0, 1) + _fshift(x, 0, -1)
    return r + _fshift(r, 1, 0) + _fshift(r, -1, 0)


def _valid(dy, dx):
    hh = lax.broadcasted_iota(jnp.int32, (1, HW), 1) // W
    ww = lax.broadcasted_iota(jnp.int32, (1, HW), 1) % W
    return (hh + dy >= 0) & (hh + dy < H) & (ww + dx >= 0) & (ww + dx < W)


def _sims_body(x_ref, sims_ref):
    ft = x_ref[0:C, :]
    fm = x_ref[C:2 * C, :]
    na = jnp.maximum(jnp.sqrt(_boxsum(jnp.sum(ft * ft, axis=0, keepdims=True))), 1e-12)
    nb = jnp.maximum(jnp.sqrt(_boxsum(jnp.sum(fm * fm, axis=0, keepdims=True))), 1e-12)
    for di in range(3):
        for dj in range(3):
            dy, dx = di - 1, dj - 1
            corr = jnp.sum(ft * _fshift(fm, dy, dx), axis=0, keepdims=True)
            raw = _boxsum(corr)
            nbs = jnp.maximum(_fshift(nb, dy, dx), 1e-12)
            sim = jnp.where(_valid(dy, dx), raw / (na * nbs), 0.0)
            sims_ref[pl.ds(di * 3 + dj, 1), :] = sim


def _topk_body(sims_hbm, awb_hbm, a_hbm, sims_v, a_v, awv):
    wid = lax.axis_index("s") * 2 + lax.axis_index("c")
    base = wid * PXW
    pltpu.sync_copy(sims_hbm.at[:, pl.ds(base, PXW)], sims_v)
    pltpu.sync_copy(awb_hbm, awv)
    for g in range(PXW // L):
        cur = [sims_v[dd, pl.ds(g * L, L)] for dd in range(D2)]
        amaps = [jnp.zeros((L,), F32) for _ in range(D2)]
        for r in range(4):
            aw_r = awv[r, :]
            best = cur[0]
            bidx = jnp.zeros((L,), jnp.int32)
            for dd in range(1, D2):
                cond = cur[dd] > best
                best = jnp.where(cond, cur[dd], best)
                bidx = jnp.where(cond, dd, bidx)
            for dd in range(D2):
                amaps[dd] = amaps[dd] + jnp.where(bidx == dd, aw_r, 0.0)
            cur = [jnp.where(bidx == dd, -5.0, cur[dd]) for dd in range(D2)]
        for dd in range(D2):
            a_v[dd, pl.ds(g * L, L)] = amaps[dd]
    pltpu.sync_copy(a_v, a_hbm.at[:, pl.ds(base, PXW)])


def _agg_body(x_ref, wp_ref, pb_ref, a_ref, ab_ref, out_ref, fms_ref):
    fm = x_ref[C:2 * C, :]
    for u in range(-2, 3):
        for v in range(-2, 3):
            fms_ref[(u + 2) * 5 + (v + 2)] = _fshift(fm, u, v)
    ab = ab_ref[0]
    a_b = []
    for di in range(3):
        for dj in range(3):
            dd = di * 3 + dj
            a_dd = a_ref[pl.ds(dd, 1), :] * _valid(di - 1, dj - 1).astype(F32)
            a_b.append(jnp.broadcast_to(a_dd, (C, HW)))
    acc = jnp.zeros((C, HW), F32)
    for pp in range(P2):
        pi, pj = pp // 3, pp % 3
        wp_pp = jnp.dot(wp_ref[pp * C:(pp + 1) * C, :], x_ref[...],
                        preferred_element_type=F32) + pb_ref[pp * C:(pp + 1) * C, :]
        b_pp = jnp.zeros((C, HW), F32)
        for di in range(3):
            for dj in range(3):
                dd = di * 3 + dj
                sel = (di - 1 + pi - 1 + 2) * 5 + (dj - 1 + pj - 1 + 2)
                b_pp = b_pp + a_b[dd] * fms_ref[sel]
        acc = acc + wp_pp * (b_pp + ab)
    out_ref[...] = acc


@functools.partial(jax.jit, static_argnames=())
def kernel(feat_t, feat_tm1, agg_w, agg_b, proj_w, proj_b):
    x = jnp.concatenate([feat_t.reshape(C, HW), feat_tm1.reshape(C, HW)], axis=0)
    wp2 = proj_w.reshape(C, P2, 2 * C).transpose(1, 0, 2).reshape(P2 * C, 2 * C)
    pb2 = proj_b.reshape(C, P2).T.reshape(P2 * C, 1)
    awb = jnp.broadcast_to(agg_w.reshape(4, 1), (4, L))
    ab = agg_b.reshape(1)

    sims = pl.pallas_call(
        _sims_body,
        out_shape=jax.ShapeDtypeStruct((D2, HW), F32),
        in_specs=[pl.BlockSpec(memory_space=pltpu.VMEM)],
    )(x)

    sc_topk = pl.kernel(
        _topk_body,
        out_type=jax.ShapeDtypeStruct((D2, HW), F32),
        mesh=plsc.VectorSubcoreMesh(core_axis_name="c", subcore_axis_name="s"),
        scratch_types=[
            pltpu.VMEM((D2, PXW), F32),
            pltpu.VMEM((D2, PXW), F32),
            pltpu.VMEM((4, L), F32),
        ],
    )
    amap = sc_topk(sims, awb)

    out = pl.pallas_call(
        _agg_body,
        out_shape=jax.ShapeDtypeStruct((C, HW), F32),
        in_specs=[
            pl.BlockSpec(memory_space=pltpu.VMEM),
            pl.BlockSpec(memory_space=pltpu.VMEM),
            pl.BlockSpec(memory_space=pltpu.VMEM),
            pl.BlockSpec(memory_space=pltpu.VMEM),
            pl.BlockSpec(memory_space=pltpu.SMEM),
        ],
        scratch_shapes=[pltpu.VMEM((25, C, HW), F32)],
    )(x, wp2, pb2, amap, ab)
    return out.reshape(1, C, H, W)
